# baseline (speedup 1.0000x reference)
import functools

import jax
import jax.numpy as jnp
from jax.experimental import pallas as pl
from jax.experimental.pallas import tpu as pltpu

_VMEM_LIMIT = 56 * 1024 * 1024

_TM_CAST = 256
_TM = 512

_PARAMS = pltpu.CompilerParams(
    dimension_semantics=("arbitrary",),
    vmem_limit_bytes=_VMEM_LIMIT,
)


def _block_stats(h, cs_ref, ss_ref):
    cs_ref[...] = jnp.sum(h, axis=0, keepdims=True)[None]
    ss_ref[...] = jnp.full((1, 1, 128), jnp.sum(h * h), jnp.float32)


def _stat_specs(gi, n):
    shapes = (jax.ShapeDtypeStruct((gi, 1, n), jnp.float32),
              jax.ShapeDtypeStruct((gi, 1, 128), jnp.float32))
    specs = (pl.BlockSpec((1, 1, n), lambda i: (i, 0, 0)),
             pl.BlockSpec((1, 1, 128), lambda i: (i, 0, 0)))
    return shapes, specs


def _layer0_kernel(x_ref, w_ref, a_ref, b_ref, abf_ref, h_ref, cs_ref,
                   ss_ref, xw_ref):
    @pl.when(pl.program_id(0) == 0)
    def _():
        xw_ref[...] = jnp.dot(
            x_ref[...].astype(jnp.bfloat16), w_ref[...].astype(jnp.bfloat16),
            preferred_element_type=jnp.float32).astype(jnp.bfloat16)

    a = a_ref[...].astype(jnp.bfloat16)
    abf_ref[...] = a
    h = jnp.dot(a, xw_ref[...], preferred_element_type=jnp.float32) + b_ref[...]
    h_ref[...] = h
    _block_stats(h, cs_ref, ss_ref)


def _layer0_call(x, w0, a, bias_row):
    m, kdim = a.shape
    f = x.shape[1]
    n = w0.shape[1]
    gi = m // _TM_CAST
    stat_shapes, stat_specs = _stat_specs(gi, n)
    return pl.pallas_call(
        _layer0_kernel,
        out_shape=(jax.ShapeDtypeStruct((m, kdim), jnp.bfloat16),
                   jax.ShapeDtypeStruct((m, n), jnp.float32)) + stat_shapes,
        grid=(gi,),
        in_specs=[
            pl.BlockSpec((m, f), lambda i: (0, 0)),
            pl.BlockSpec((f, n), lambda i: (0, 0)),
            pl.BlockSpec((_TM_CAST, kdim), lambda i: (i, 0)),
            pl.BlockSpec((1, n), lambda i: (0, 0)),
        ],
        out_specs=(pl.BlockSpec((_TM_CAST, kdim), lambda i: (i, 0)),
                   pl.BlockSpec((_TM_CAST, n), lambda i: (i, 0))) + stat_specs,
        scratch_shapes=[pltpu.VMEM((kdim, n), jnp.bfloat16)],
        compiler_params=_PARAMS,
    )(x, w0, a, bias_row)


def _layerN_kernel(h_prev_ref, cs_in_ref, ss_in_ref, w_ref, a_ref, b_ref,
                   h_ref, cs_ref, ss_ref, xw_ref, *, n_nodes, with_stats):
    @pl.when(pl.program_id(0) == 0)
    def _():
        n = jnp.float32(n_nodes)
        col_mean = jnp.sum(cs_in_ref[...], axis=0) / n
        sumsq = jnp.sum(ss_in_ref[:, :, 0])
        total_sq = sumsq - n * jnp.sum(col_mean * col_mean)
        inv = jax.lax.rsqrt(1e-6 + total_sq / n)
        y = (h_prev_ref[...] - col_mean) * inv
        y = jnp.maximum(y, 0.0).astype(jnp.bfloat16)
        xw_ref[...] = jnp.dot(
            y, w_ref[...].astype(jnp.bfloat16),
            preferred_element_type=jnp.float32).astype(jnp.bfloat16)

    h = jnp.dot(a_ref[...], xw_ref[...],
                preferred_element_type=jnp.float32) + b_ref[...]
    h_ref[...] = h
    if with_stats:
        _block_stats(h, cs_ref, ss_ref)


def _layerN_call(h_prev, cs_in, ss_in, w, a_bf, bias_row, n_nodes,
                 with_stats):
    m, kdim = a_bf.shape
    d = h_prev.shape[1]
    n = w.shape[1]
    gi = m // _TM
    gprev = cs_in.shape[0]
    kern = functools.partial(_layerN_kernel, n_nodes=n_nodes,
                             with_stats=with_stats)
    in_specs = [
        pl.BlockSpec((m, d), lambda i: (0, 0)),
        pl.BlockSpec((gprev, 1, d), lambda i: (0, 0, 0)),
        pl.BlockSpec((gprev, 1, 128), lambda i: (0, 0, 0)),
        pl.BlockSpec((d, n), lambda i: (0, 0)),
        pl.BlockSpec((_TM, kdim), lambda i: (i, 0)),
        pl.BlockSpec((1, n), lambda i: (0, 0)),
    ]
    h_shape = jax.ShapeDtypeStruct((m, n), jnp.float32)
    h_spec = pl.BlockSpec((_TM, n), lambda i: (i, 0))
    scratch = [pltpu.VMEM((kdim, n), jnp.bfloat16)]
    if with_stats:
        stat_shapes, stat_specs = _stat_specs(gi, n)
        return pl.pallas_call(
            kern,
            out_shape=(h_shape,) + stat_shapes,
            grid=(gi,),
            in_specs=in_specs,
            out_specs=(h_spec,) + stat_specs,
            scratch_shapes=scratch,
            compiler_params=_PARAMS,
        )(h_prev, cs_in, ss_in, w, a_bf, bias_row)

    def _kern_plain(h_prev_ref, cs_in_ref, ss_in_ref, w_ref, a_ref, b_ref,
                    h_ref, xw_ref):
        kern(h_prev_ref, cs_in_ref, ss_in_ref, w_ref, a_ref, b_ref,
             h_ref, None, None, xw_ref)

    return pl.pallas_call(
        _kern_plain,
        out_shape=h_shape,
        grid=(gi,),
        in_specs=in_specs,
        out_specs=h_spec,
        scratch_shapes=scratch,
        compiler_params=_PARAMS,
    )(h_prev, cs_in, ss_in, w, a_bf, bias_row)


def kernel(x, a_hat, w0, w1, w2, b0, b1, b2):
    n_nodes = x.shape[0]
    d_out = w2.shape[1]
    d_out_p = 128

    w2p = jnp.pad(w2, ((0, 0), (0, d_out_p - d_out)))
    b0r = b0.reshape(1, -1)
    b1r = b1.reshape(1, -1)
    b2r = jnp.pad(b2.reshape(1, -1), ((0, 0), (0, d_out_p - d_out)))

    a_bf, h0, cs0, ss0 = _layer0_call(x, w0, a_hat, b0r)
    h1, cs1, ss1 = _layerN_call(h0, cs0, ss0, w1, a_bf, b1r, n_nodes, True)
    out = _layerN_call(h1, cs1, ss1, w2p, a_bf, b2r, n_nodes, False)
    return out[:, :d_out]


# speedup vs baseline: 1.0223x; 1.0223x over previous
import functools

import jax
import jax.numpy as jnp
from jax.experimental import pallas as pl
from jax.experimental.pallas import tpu as pltpu

_VMEM_LIMIT = 56 * 1024 * 1024

_TM_CAST = 256
_TM = 1024

_PARAMS = pltpu.CompilerParams(
    dimension_semantics=("arbitrary",),
    vmem_limit_bytes=_VMEM_LIMIT,
)


def _block_stats(h, cs_ref, ss_ref):
    cs_ref[...] = jnp.sum(h, axis=0, keepdims=True)[None]
    ss_ref[...] = jnp.full((1, 1, 128), jnp.sum(h * h), jnp.float32)


def _stat_specs(gi, n):
    shapes = (jax.ShapeDtypeStruct((gi, 1, n), jnp.float32),
              jax.ShapeDtypeStruct((gi, 1, 128), jnp.float32))
    specs = (pl.BlockSpec((1, 1, n), lambda i: (i, 0, 0)),
             pl.BlockSpec((1, 1, 128), lambda i: (i, 0, 0)))
    return shapes, specs


def _layer0_kernel(x_ref, w_ref, a_ref, b_ref, abf_ref, h_ref, cs_ref,
                   ss_ref, xw_ref):
    @pl.when(pl.program_id(0) == 0)
    def _():
        xw_ref[...] = jnp.dot(
            x_ref[...].astype(jnp.bfloat16), w_ref[...].astype(jnp.bfloat16),
            preferred_element_type=jnp.float32).astype(jnp.bfloat16)

    a = a_ref[...].astype(jnp.bfloat16)
    abf_ref[...] = a
    h = jnp.dot(a, xw_ref[...], preferred_element_type=jnp.float32) + b_ref[...]
    h_ref[...] = h
    _block_stats(h, cs_ref, ss_ref)


def _layer0_call(x, w0, a, bias_row):
    m, kdim = a.shape
    f = x.shape[1]
    n = w0.shape[1]
    gi = m // _TM_CAST
    stat_shapes, stat_specs = _stat_specs(gi, n)
    return pl.pallas_call(
        _layer0_kernel,
        out_shape=(jax.ShapeDtypeStruct((m, kdim), jnp.bfloat16),
                   jax.ShapeDtypeStruct((m, n), jnp.float32)) + stat_shapes,
        grid=(gi,),
        in_specs=[
            pl.BlockSpec((m, f), lambda i: (0, 0)),
            pl.BlockSpec((f, n), lambda i: (0, 0)),
            pl.BlockSpec((_TM_CAST, kdim), lambda i: (i, 0)),
            pl.BlockSpec((1, n), lambda i: (0, 0)),
        ],
        out_specs=(pl.BlockSpec((_TM_CAST, kdim), lambda i: (i, 0)),
                   pl.BlockSpec((_TM_CAST, n), lambda i: (i, 0))) + stat_specs,
        scratch_shapes=[pltpu.VMEM((kdim, n), jnp.bfloat16)],
        compiler_params=_PARAMS,
    )(x, w0, a, bias_row)


def _layerN_kernel(h_prev_ref, cs_in_ref, ss_in_ref, w_ref, a_ref, b_ref,
                   h_ref, cs_ref, ss_ref, xw_ref, *, n_nodes, with_stats):
    @pl.when(pl.program_id(0) == 0)
    def _():
        n = jnp.float32(n_nodes)
        col_mean = jnp.sum(cs_in_ref[...], axis=0) / n
        sumsq = jnp.sum(ss_in_ref[:, :, 0])
        total_sq = sumsq - n * jnp.sum(col_mean * col_mean)
        inv = jax.lax.rsqrt(1e-6 + total_sq / n)
        y = (h_prev_ref[...] - col_mean) * inv
        y = jnp.maximum(y, 0.0).astype(jnp.bfloat16)
        xw_ref[...] = jnp.dot(
            y, w_ref[...].astype(jnp.bfloat16),
            preferred_element_type=jnp.float32).astype(jnp.bfloat16)

    h = jnp.dot(a_ref[...], xw_ref[...],
                preferred_element_type=jnp.float32) + b_ref[...]
    h_ref[...] = h
    if with_stats:
        _block_stats(h, cs_ref, ss_ref)


def _layerN_call(h_prev, cs_in, ss_in, w, a_bf, bias_row, n_nodes,
                 with_stats):
    m, kdim = a_bf.shape
    d = h_prev.shape[1]
    n = w.shape[1]
    gi = m // _TM
    gprev = cs_in.shape[0]
    kern = functools.partial(_layerN_kernel, n_nodes=n_nodes,
                             with_stats=with_stats)
    in_specs = [
        pl.BlockSpec((m, d), lambda i: (0, 0)),
        pl.BlockSpec((gprev, 1, d), lambda i: (0, 0, 0)),
        pl.BlockSpec((gprev, 1, 128), lambda i: (0, 0, 0)),
        pl.BlockSpec((d, n), lambda i: (0, 0)),
        pl.BlockSpec((_TM, kdim), lambda i: (i, 0)),
        pl.BlockSpec((1, n), lambda i: (0, 0)),
    ]
    h_shape = jax.ShapeDtypeStruct((m, n), jnp.float32)
    h_spec = pl.BlockSpec((_TM, n), lambda i: (i, 0))
    scratch = [pltpu.VMEM((kdim, n), jnp.bfloat16)]
    if with_stats:
        stat_shapes, stat_specs = _stat_specs(gi, n)
        return pl.pallas_call(
            kern,
            out_shape=(h_shape,) + stat_shapes,
            grid=(gi,),
            in_specs=in_specs,
            out_specs=(h_spec,) + stat_specs,
            scratch_shapes=scratch,
            compiler_params=_PARAMS,
        )(h_prev, cs_in, ss_in, w, a_bf, bias_row)

    def _kern_plain(h_prev_ref, cs_in_ref, ss_in_ref, w_ref, a_ref, b_ref,
                    h_ref, xw_ref):
        kern(h_prev_ref, cs_in_ref, ss_in_ref, w_ref, a_ref, b_ref,
             h_ref, None, None, xw_ref)

    return pl.pallas_call(
        _kern_plain,
        out_shape=h_shape,
        grid=(gi,),
        in_specs=in_specs,
        out_specs=h_spec,
        scratch_shapes=scratch,
        compiler_params=_PARAMS,
    )(h_prev, cs_in, ss_in, w, a_bf, bias_row)


def kernel(x, a_hat, w0, w1, w2, b0, b1, b2):
    n_nodes = x.shape[0]
    d_out = w2.shape[1]
    d_out_p = 128

    w2p = jnp.pad(w2, ((0, 0), (0, d_out_p - d_out)))
    b0r = b0.reshape(1, -1)
    b1r = b1.reshape(1, -1)
    b2r = jnp.pad(b2.reshape(1, -1), ((0, 0), (0, d_out_p - d_out)))

    a_bf, h0, cs0, ss0 = _layer0_call(x, w0, a_hat, b0r)
    h1, cs1, ss1 = _layerN_call(h0, cs0, ss0, w1, a_bf, b1r, n_nodes, True)
    out = _layerN_call(h1, cs1, ss1, w2p, a_bf, b2r, n_nodes, False)
    return out[:, :d_out]


# speedup vs baseline: 1.0378x; 1.0151x over previous
import functools

import jax
import jax.numpy as jnp
from jax.experimental import pallas as pl
from jax.experimental.pallas import tpu as pltpu

_VMEM_LIMIT = 56 * 1024 * 1024

_TM_CAST = 256
_TM = 1024

_PARAMS = pltpu.CompilerParams(
    dimension_semantics=("arbitrary",),
    vmem_limit_bytes=_VMEM_LIMIT,
)


def _block_stats(h, cs_ref, ss_ref):
    cs_ref[...] = jnp.sum(h, axis=0, keepdims=True)[None]
    ss_ref[...] = jnp.full((1, 1, 128), jnp.sum(h * h), jnp.float32)


def _stat_specs(gi, n):
    shapes = (jax.ShapeDtypeStruct((gi, 1, n), jnp.float32),
              jax.ShapeDtypeStruct((gi, 1, 128), jnp.float32))
    specs = (pl.BlockSpec((1, 1, n), lambda i: (i, 0, 0)),
             pl.BlockSpec((1, 1, 128), lambda i: (i, 0, 0)))
    return shapes, specs


def _layer0_kernel(x_ref, w_ref, a_ref, b_ref, abf_ref, h_ref, cs_ref,
                   ss_ref, xw_ref):
    @pl.when(pl.program_id(0) == 0)
    def _():
        xw_ref[...] = jnp.dot(
            x_ref[...].astype(jnp.bfloat16), w_ref[...].astype(jnp.bfloat16),
            preferred_element_type=jnp.float32).astype(jnp.bfloat16)

    a = a_ref[...].astype(jnp.bfloat16)
    abf_ref[...] = a
    h = jnp.dot(a, xw_ref[...], preferred_element_type=jnp.float32) + b_ref[...]
    h_ref[...] = h.astype(h_ref.dtype)
    _block_stats(h, cs_ref, ss_ref)


def _layer0_call(x, w0, a, bias_row):
    m, kdim = a.shape
    f = x.shape[1]
    n = w0.shape[1]
    gi = m // _TM_CAST
    stat_shapes, stat_specs = _stat_specs(gi, n)
    return pl.pallas_call(
        _layer0_kernel,
        out_shape=(jax.ShapeDtypeStruct((m, kdim), jnp.bfloat16),
                   jax.ShapeDtypeStruct((m, n), jnp.bfloat16)) + stat_shapes,
        grid=(gi,),
        in_specs=[
            pl.BlockSpec((m, f), lambda i: (0, 0)),
            pl.BlockSpec((f, n), lambda i: (0, 0)),
            pl.BlockSpec((_TM_CAST, kdim), lambda i: (i, 0)),
            pl.BlockSpec((1, n), lambda i: (0, 0)),
        ],
        out_specs=(pl.BlockSpec((_TM_CAST, kdim), lambda i: (i, 0)),
                   pl.BlockSpec((_TM_CAST, n), lambda i: (i, 0))) + stat_specs,
        scratch_shapes=[pltpu.VMEM((kdim, n), jnp.bfloat16)],
        compiler_params=_PARAMS,
    )(x, w0, a, bias_row)


def _layerN_kernel(h_prev_ref, cs_in_ref, ss_in_ref, w_ref, a_ref, b_ref,
                   h_ref, cs_ref, ss_ref, xw_ref, *, n_nodes, with_stats):
    @pl.when(pl.program_id(0) == 0)
    def _():
        n = jnp.float32(n_nodes)
        col_mean = jnp.sum(cs_in_ref[...], axis=0) / n
        sumsq = jnp.sum(ss_in_ref[:, :, 0])
        total_sq = sumsq - n * jnp.sum(col_mean * col_mean)
        inv = jax.lax.rsqrt(1e-6 + total_sq / n)
        y = (h_prev_ref[...].astype(jnp.float32) - col_mean) * inv
        y = jnp.maximum(y, 0.0).astype(jnp.bfloat16)
        xw_ref[...] = jnp.dot(
            y, w_ref[...].astype(jnp.bfloat16),
            preferred_element_type=jnp.float32).astype(jnp.bfloat16)

    h = jnp.dot(a_ref[...], xw_ref[...],
                preferred_element_type=jnp.float32) + b_ref[...]
    h_ref[...] = h[:, :h_ref.shape[-1]].astype(h_ref.dtype)
    if with_stats:
        _block_stats(h, cs_ref, ss_ref)


def _layerN_call(h_prev, cs_in, ss_in, w, a_bf, bias_row, n_nodes,
                 with_stats, n_out=None):
    m, kdim = a_bf.shape
    d = h_prev.shape[1]
    n = w.shape[1]
    gi = m // _TM
    gprev = cs_in.shape[0]
    kern = functools.partial(_layerN_kernel, n_nodes=n_nodes,
                             with_stats=with_stats)
    in_specs = [
        pl.BlockSpec((m, d), lambda i: (0, 0)),
        pl.BlockSpec((gprev, 1, d), lambda i: (0, 0, 0)),
        pl.BlockSpec((gprev, 1, 128), lambda i: (0, 0, 0)),
        pl.BlockSpec((d, n), lambda i: (0, 0)),
        pl.BlockSpec((_TM, kdim), lambda i: (i, 0)),
        pl.BlockSpec((1, n), lambda i: (0, 0)),
    ]
    scratch = [pltpu.VMEM((kdim, n), jnp.bfloat16)]
    if with_stats:
        h_shape = jax.ShapeDtypeStruct((m, n), jnp.bfloat16)
        h_spec = pl.BlockSpec((_TM, n), lambda i: (i, 0))
        stat_shapes, stat_specs = _stat_specs(gi, n)
        return pl.pallas_call(
            kern,
            out_shape=(h_shape,) + stat_shapes,
            grid=(gi,),
            in_specs=in_specs,
            out_specs=(h_spec,) + stat_specs,
            scratch_shapes=scratch,
            compiler_params=_PARAMS,
        )(h_prev, cs_in, ss_in, w, a_bf, bias_row)

    def _kern_plain(h_prev_ref, cs_in_ref, ss_in_ref, w_ref, a_ref, b_ref,
                    h_ref, xw_ref):
        kern(h_prev_ref, cs_in_ref, ss_in_ref, w_ref, a_ref, b_ref,
             h_ref, None, None, xw_ref)

    return pl.pallas_call(
        _kern_plain,
        out_shape=jax.ShapeDtypeStruct((m, n_out), jnp.float32),
        grid=(gi,),
        in_specs=in_specs,
        out_specs=pl.BlockSpec((_TM, n_out), lambda i: (i, 0)),
        scratch_shapes=scratch,
        compiler_params=_PARAMS,
    )(h_prev, cs_in, ss_in, w, a_bf, bias_row)


def kernel(x, a_hat, w0, w1, w2, b0, b1, b2):
    n_nodes = x.shape[0]
    d_out = w2.shape[1]
    d_out_p = 128

    w2p = jnp.pad(w2, ((0, 0), (0, d_out_p - d_out)))
    b0r = b0.reshape(1, -1)
    b1r = b1.reshape(1, -1)
    b2r = jnp.pad(b2.reshape(1, -1), ((0, 0), (0, d_out_p - d_out)))

    a_bf, h0, cs0, ss0 = _layer0_call(x, w0, a_hat, b0r)
    h1, cs1, ss1 = _layerN_call(h0, cs0, ss0, w1, a_bf, b1r, n_nodes, True)
    return _layerN_call(h1, cs1, ss1, w2p, a_bf, b2r, n_nodes, False,
                        n_out=d_out)


# speedup vs baseline: 1.0482x; 1.0100x over previous
import functools

import jax
import jax.numpy as jnp
from jax.experimental import pallas as pl
from jax.experimental.pallas import tpu as pltpu

_VMEM_LIMIT = 58 * 1024 * 1024

_TM_CAST = 512
_TM = 1024

_PARAMS = pltpu.CompilerParams(
    dimension_semantics=("arbitrary",),
    vmem_limit_bytes=_VMEM_LIMIT,
)


def _block_stats(h, cs_ref, ss_ref):
    cs_ref[...] = jnp.sum(h, axis=0, keepdims=True)[None]
    ss_ref[...] = jnp.full((1, 1, 128), jnp.sum(h * h), jnp.float32)


def _stat_specs(gi, n):
    shapes = (jax.ShapeDtypeStruct((gi, 1, n), jnp.float32),
              jax.ShapeDtypeStruct((gi, 1, 128), jnp.float32))
    specs = (pl.BlockSpec((1, 1, n), lambda i: (i, 0, 0)),
             pl.BlockSpec((1, 1, 128), lambda i: (i, 0, 0)))
    return shapes, specs


def _layer0_kernel(x_ref, w_ref, a_ref, b_ref, abf_ref, h_ref, cs_ref,
                   ss_ref, xw_ref):
    @pl.when(pl.program_id(0) == 0)
    def _():
        xw_ref[...] = jnp.dot(
            x_ref[...].astype(jnp.bfloat16), w_ref[...].astype(jnp.bfloat16),
            preferred_element_type=jnp.float32).astype(jnp.bfloat16)

    a = a_ref[...].astype(jnp.bfloat16)
    abf_ref[...] = a
    h = jnp.dot(a, xw_ref[...], preferred_element_type=jnp.float32) + b_ref[...]
    h_ref[...] = h.astype(h_ref.dtype)
    _block_stats(h, cs_ref, ss_ref)


def _layer0_call(x, w0, a, bias_row):
    m, kdim = a.shape
    f = x.shape[1]
    n = w0.shape[1]
    gi = m // _TM_CAST
    stat_shapes, stat_specs = _stat_specs(gi, n)
    return pl.pallas_call(
        _layer0_kernel,
        out_shape=(jax.ShapeDtypeStruct((m, kdim), jnp.bfloat16),
                   jax.ShapeDtypeStruct((m, n), jnp.bfloat16)) + stat_shapes,
        grid=(gi,),
        in_specs=[
            pl.BlockSpec((m, f), lambda i: (0, 0)),
            pl.BlockSpec((f, n), lambda i: (0, 0)),
            pl.BlockSpec((_TM_CAST, kdim), lambda i: (i, 0)),
            pl.BlockSpec((1, n), lambda i: (0, 0)),
        ],
        out_specs=(pl.BlockSpec((_TM_CAST, kdim), lambda i: (i, 0)),
                   pl.BlockSpec((_TM_CAST, n), lambda i: (i, 0))) + stat_specs,
        scratch_shapes=[pltpu.VMEM((kdim, n), jnp.bfloat16)],
        compiler_params=_PARAMS,
    )(x, w0, a, bias_row)


def _layerN_kernel(h_prev_ref, cs_in_ref, ss_in_ref, w_ref, a_ref, b_ref,
                   h_ref, cs_ref, ss_ref, xw_ref, *, n_nodes, with_stats):
    @pl.when(pl.program_id(0) == 0)
    def _():
        n = jnp.float32(n_nodes)
        col_mean = jnp.sum(cs_in_ref[...], axis=0) / n
        sumsq = jnp.sum(ss_in_ref[:, :, 0])
        total_sq = sumsq - n * jnp.sum(col_mean * col_mean)
        inv = jax.lax.rsqrt(1e-6 + total_sq / n)
        y = (h_prev_ref[...].astype(jnp.float32) - col_mean) * inv
        y = jnp.maximum(y, 0.0).astype(jnp.bfloat16)
        xw_ref[...] = jnp.dot(
            y, w_ref[...].astype(jnp.bfloat16),
            preferred_element_type=jnp.float32).astype(jnp.bfloat16)

    h = jnp.dot(a_ref[...], xw_ref[...],
                preferred_element_type=jnp.float32) + b_ref[...]
    h_ref[...] = h[:, :h_ref.shape[-1]].astype(h_ref.dtype)
    if with_stats:
        _block_stats(h, cs_ref, ss_ref)


def _layerN_call(h_prev, cs_in, ss_in, w, a_bf, bias_row, n_nodes,
                 with_stats, n_out=None):
    m, kdim = a_bf.shape
    d = h_prev.shape[1]
    n = w.shape[1]
    gi = m // _TM
    gprev = cs_in.shape[0]
    kern = functools.partial(_layerN_kernel, n_nodes=n_nodes,
                             with_stats=with_stats)
    in_specs = [
        pl.BlockSpec((m, d), lambda i: (0, 0)),
        pl.BlockSpec((gprev, 1, d), lambda i: (0, 0, 0)),
        pl.BlockSpec((gprev, 1, 128), lambda i: (0, 0, 0)),
        pl.BlockSpec((d, n), lambda i: (0, 0)),
        pl.BlockSpec((_TM, kdim), lambda i: (i, 0)),
        pl.BlockSpec((1, n), lambda i: (0, 0)),
    ]
    scratch = [pltpu.VMEM((kdim, n), jnp.bfloat16)]
    if with_stats:
        h_shape = jax.ShapeDtypeStruct((m, n), jnp.bfloat16)
        h_spec = pl.BlockSpec((_TM, n), lambda i: (i, 0))
        stat_shapes, stat_specs = _stat_specs(gi, n)
        return pl.pallas_call(
            kern,
            out_shape=(h_shape,) + stat_shapes,
            grid=(gi,),
            in_specs=in_specs,
            out_specs=(h_spec,) + stat_specs,
            scratch_shapes=scratch,
            compiler_params=_PARAMS,
        )(h_prev, cs_in, ss_in, w, a_bf, bias_row)

    def _kern_plain(h_prev_ref, cs_in_ref, ss_in_ref, w_ref, a_ref, b_ref,
                    h_ref, xw_ref):
        kern(h_prev_ref, cs_in_ref, ss_in_ref, w_ref, a_ref, b_ref,
             h_ref, None, None, xw_ref)

    return pl.pallas_call(
        _kern_plain,
        out_shape=jax.ShapeDtypeStruct((m, n_out), jnp.float32),
        grid=(gi,),
        in_specs=in_specs,
        out_specs=pl.BlockSpec((_TM, n_out), lambda i: (i, 0)),
        scratch_shapes=scratch,
        compiler_params=_PARAMS,
    )(h_prev, cs_in, ss_in, w, a_bf, bias_row)


def kernel(x, a_hat, w0, w1, w2, b0, b1, b2):
    n_nodes = x.shape[0]
    d_out = w2.shape[1]
    d_out_p = 128

    w2p = jnp.pad(w2, ((0, 0), (0, d_out_p - d_out)))
    b0r = b0.reshape(1, -1)
    b1r = b1.reshape(1, -1)
    b2r = jnp.pad(b2.reshape(1, -1), ((0, 0), (0, d_out_p - d_out)))

    a_bf, h0, cs0, ss0 = _layer0_call(x, w0, a_hat, b0r)
    h1, cs1, ss1 = _layerN_call(h0, cs0, ss0, w1, a_bf, b1r, n_nodes, True)
    return _layerN_call(h1, cs1, ss1, w2p, a_bf, b2r, n_nodes, False,
                        n_out=d_out)
